# baseline (speedup 1.0000x reference)
import math

import jax
import jax.numpy as jnp
from jax.experimental import pallas as pl
from jax.experimental.pallas import tpu as pltpu

D_MODEL = 1024
D_CACHE = 256
INV_SQRT_DC = 1.0 / math.sqrt(D_CACHE)


def _proj_kernel(x_ref, W_to_ref, b_to_ref, xc_ref):
    xc_ref[...] = (jnp.dot(x_ref[...], W_to_ref[...],
                           preferred_element_type=jnp.float32)
                   + b_to_ref[...])


def _attend_kernel(xc_ref, cache_ref, ctx_ref):
    xcb = xc_ref[0]
    cb = cache_ref[0]
    scores = jax.lax.dot_general(
        xcb, cb, (((1,), (1,)), ((), ())),
        preferred_element_type=jnp.float32)
    u = jnp.exp(jnp.clip(scores * INV_SQRT_DC, -20.0, 20.0))
    ctx_u = jnp.dot(u, cb, preferred_element_type=jnp.float32)
    denom = jnp.sum(u, axis=-1, keepdims=True)
    ctx_ref[0] = ctx_u / denom


def _gate_kernel(x_ref, xc_ref, ctx_ref, ln_g_ref, ln_b_ref,
                 W1_ref, b1_ref, W2r_ref, b2_ref, W_from_ref, b_from_ref,
                 out_ref):
    xc = xc_ref[...]
    ctx = ctx_ref[...]
    comb = jnp.concatenate([xc, ctx], axis=-1)
    mean = jnp.mean(comb, axis=-1, keepdims=True)
    var = jnp.mean((comb - mean) ** 2, axis=-1, keepdims=True)
    h = ln_g_ref[...] * (comb - mean) / jnp.sqrt(var + 1e-5) + ln_b_ref[...]
    h = jnp.dot(h, W1_ref[...], preferred_element_type=jnp.float32) + b1_ref[...]
    h = h * jax.nn.sigmoid(h)
    logit = jnp.sum(h * W2r_ref[...], axis=-1, keepdims=True) + b2_ref[...]
    gate = (logit > 0.0).astype(jnp.float32)
    ctx_d = jnp.dot(ctx, W_from_ref[...], preferred_element_type=jnp.float32)
    out_ref[...] = x_ref[...] + gate * (ctx_d + b_from_ref[...])


def kernel(x, cache, W_to, b_to, ln_g, ln_b, W1, b1, W2, b2, W_from, b_from):
    B, S, _ = x.shape
    M = cache.shape[1]
    T = B * S
    x2 = x.reshape(T, D_MODEL)
    b_to2 = b_to.reshape(1, D_CACHE)
    ln_g2 = ln_g.reshape(1, 2 * D_CACHE)
    ln_b2 = ln_b.reshape(1, 2 * D_CACHE)
    b12 = b1.reshape(1, D_MODEL // 2)
    W2r = W2.reshape(1, D_MODEL // 2)
    b22 = b2.reshape(1, 1)
    b_from2 = b_from.reshape(1, D_MODEL)

    full = lambda shape: pl.BlockSpec(shape, lambda *a: (0,) * len(shape))

    xc = pl.pallas_call(
        _proj_kernel,
        in_specs=[full((T, D_MODEL)), full((D_MODEL, D_CACHE)),
                  full((1, D_CACHE))],
        out_specs=full((T, D_CACHE)),
        out_shape=jax.ShapeDtypeStruct((T, D_CACHE), jnp.float32),
    )(x2, W_to, b_to2)

    ctx = pl.pallas_call(
        _attend_kernel,
        grid=(B,),
        in_specs=[
            pl.BlockSpec((1, S, D_CACHE), lambda b: (b, 0, 0)),
            pl.BlockSpec((1, M, D_CACHE), lambda b: (b, 0, 0)),
        ],
        out_specs=pl.BlockSpec((1, S, D_CACHE), lambda b: (b, 0, 0)),
        out_shape=jax.ShapeDtypeStruct((B, S, D_CACHE), jnp.float32),
        compiler_params=pltpu.CompilerParams(
            dimension_semantics=("parallel",)),
    )(xc.reshape(B, S, D_CACHE), cache)

    out = pl.pallas_call(
        _gate_kernel,
        in_specs=[full((T, D_MODEL)), full((T, D_CACHE)), full((T, D_CACHE)),
                  full((1, 2 * D_CACHE)), full((1, 2 * D_CACHE)),
                  full((2 * D_CACHE, D_MODEL // 2)), full((1, D_MODEL // 2)),
                  full((1, D_MODEL // 2)), full((1, 1)),
                  full((D_CACHE, D_MODEL)), full((1, D_MODEL))],
        out_specs=full((T, D_MODEL)),
        out_shape=jax.ShapeDtypeStruct((T, D_MODEL), jnp.float32),
    )(x2, xc, ctx.reshape(T, D_CACHE), ln_g2, ln_b2,
      W1, b12, W2r, b22, W_from, b_from2)
    return out.reshape(B, S, D_MODEL)


# speedup vs baseline: 1.2151x; 1.2151x over previous
import math

import jax
import jax.numpy as jnp
from jax.experimental import pallas as pl
from jax.experimental.pallas import tpu as pltpu

D_MODEL = 1024
D_CACHE = 256
INV_SQRT_DC = 1.0 / math.sqrt(D_CACHE)


def _proj_kernel(x_ref, W_to_ref, b_to_ref, xc_ref):
    xc_ref[...] = (jnp.dot(x_ref[...], W_to_ref[...],
                           preferred_element_type=jnp.float32)
                   + b_to_ref[...])


ATTEND_BATCHES = 2


def _attend_kernel(xc_ref, cache_ref, ctx_ref):
    for i in range(ATTEND_BATCHES):
        xcb = xc_ref[i]
        cb = cache_ref[i]
        scores = jax.lax.dot_general(
            xcb, cb, (((1,), (1,)), ((), ())),
            preferred_element_type=jnp.float32)
        u = jnp.exp(jnp.clip(scores * INV_SQRT_DC, -20.0, 20.0))
        ctx_u = jnp.dot(u, cb, preferred_element_type=jnp.float32)
        denom = jnp.sum(u, axis=-1, keepdims=True)
        ctx_ref[i] = ctx_u / denom


def _gate_kernel(x_ref, xc_ref, ctx_ref, ln_g_ref, ln_b_ref,
                 W1_ref, b1_ref, W2r_ref, b2_ref, W_from_ref, b_from_ref,
                 out_ref):
    xc = xc_ref[...]
    ctx = ctx_ref[...]
    comb = jnp.concatenate([xc, ctx], axis=-1)
    mean = jnp.mean(comb, axis=-1, keepdims=True)
    var = jnp.mean((comb - mean) ** 2, axis=-1, keepdims=True)
    h = ln_g_ref[...] * (comb - mean) / jnp.sqrt(var + 1e-5) + ln_b_ref[...]
    h = jnp.dot(h, W1_ref[...], preferred_element_type=jnp.float32) + b1_ref[...]
    h = h * jax.nn.sigmoid(h)
    logit = jnp.sum(h * W2r_ref[...], axis=-1, keepdims=True) + b2_ref[...]
    gate = (logit > 0.0).astype(jnp.float32)
    ctx_d = jnp.dot(ctx, W_from_ref[...], preferred_element_type=jnp.float32)
    out_ref[...] = x_ref[...] + gate * (ctx_d + b_from_ref[...])


def kernel(x, cache, W_to, b_to, ln_g, ln_b, W1, b1, W2, b2, W_from, b_from):
    B, S, _ = x.shape
    M = cache.shape[1]
    T = B * S
    x2 = x.reshape(T, D_MODEL)
    b_to2 = b_to.reshape(1, D_CACHE)
    ln_g2 = ln_g.reshape(1, 2 * D_CACHE)
    ln_b2 = ln_b.reshape(1, 2 * D_CACHE)
    b12 = b1.reshape(1, D_MODEL // 2)
    W2r = W2.reshape(1, D_MODEL // 2)
    b22 = b2.reshape(1, 1)
    b_from2 = b_from.reshape(1, D_MODEL)

    full = lambda shape: pl.BlockSpec(shape, lambda *a: (0,) * len(shape))

    xc = pl.pallas_call(
        _proj_kernel,
        in_specs=[full((T, D_MODEL)), full((D_MODEL, D_CACHE)),
                  full((1, D_CACHE))],
        out_specs=full((T, D_CACHE)),
        out_shape=jax.ShapeDtypeStruct((T, D_CACHE), jnp.float32),
    )(x2, W_to, b_to2)

    ctx = pl.pallas_call(
        _attend_kernel,
        grid=(B // ATTEND_BATCHES,),
        in_specs=[
            pl.BlockSpec((ATTEND_BATCHES, S, D_CACHE), lambda b: (b, 0, 0)),
            pl.BlockSpec((ATTEND_BATCHES, M, D_CACHE), lambda b: (b, 0, 0)),
        ],
        out_specs=pl.BlockSpec((ATTEND_BATCHES, S, D_CACHE), lambda b: (b, 0, 0)),
        out_shape=jax.ShapeDtypeStruct((B, S, D_CACHE), jnp.float32),
        compiler_params=pltpu.CompilerParams(
            dimension_semantics=("parallel",)),
    )(xc.reshape(B, S, D_CACHE), cache)

    out = pl.pallas_call(
        _gate_kernel,
        in_specs=[full((T, D_MODEL)), full((T, D_CACHE)), full((T, D_CACHE)),
                  full((1, 2 * D_CACHE)), full((1, 2 * D_CACHE)),
                  full((2 * D_CACHE, D_MODEL // 2)), full((1, D_MODEL // 2)),
                  full((1, D_MODEL // 2)), full((1, 1)),
                  full((D_CACHE, D_MODEL)), full((1, D_MODEL))],
        out_specs=full((T, D_MODEL)),
        out_shape=jax.ShapeDtypeStruct((T, D_MODEL), jnp.float32),
    )(x2, xc, ctx.reshape(T, D_CACHE), ln_g2, ln_b2,
      W1, b12, W2r, b22, W_from, b_from2)
    return out.reshape(B, S, D_MODEL)


# speedup vs baseline: 1.2920x; 1.0633x over previous
import math

import jax
import jax.numpy as jnp
from jax.experimental import pallas as pl
from jax.experimental.pallas import tpu as pltpu

D_MODEL = 1024
D_CACHE = 256
INV_SQRT_DC = 1.0 / math.sqrt(D_CACHE)


def _proj_kernel(x_ref, W_to_ref, b_to_ref, xc_ref):
    xc_ref[...] = (jnp.dot(x_ref[...], W_to_ref[...],
                           preferred_element_type=jnp.float32)
                   + b_to_ref[...])


ATTEND_BATCHES = 4


def _attend_kernel(xc_ref, cache_ref, ctx_ref):
    for i in range(ATTEND_BATCHES):
        xcb = xc_ref[i]
        cb = cache_ref[i]
        scores = jax.lax.dot_general(
            xcb, cb, (((1,), (1,)), ((), ())),
            preferred_element_type=jnp.float32)
        u = jnp.exp(jnp.clip(scores * INV_SQRT_DC, -20.0, 20.0))
        ctx_u = jnp.dot(u, cb, preferred_element_type=jnp.float32)
        denom = jnp.sum(u, axis=-1, keepdims=True)
        ctx_ref[i] = ctx_u / denom


def _gate_kernel(x_ref, xc_ref, ctx_ref, ln_g_ref, ln_b_ref,
                 W1_ref, b1_ref, W2r_ref, b2_ref, W_from_ref, b_from_ref,
                 out_ref):
    xc = xc_ref[...]
    ctx = ctx_ref[...]
    comb = jnp.concatenate([xc, ctx], axis=-1)
    mean = jnp.mean(comb, axis=-1, keepdims=True)
    var = jnp.mean((comb - mean) ** 2, axis=-1, keepdims=True)
    h = ln_g_ref[...] * (comb - mean) / jnp.sqrt(var + 1e-5) + ln_b_ref[...]
    h = jnp.dot(h, W1_ref[...], preferred_element_type=jnp.float32) + b1_ref[...]
    h = h * jax.nn.sigmoid(h)
    logit = jnp.sum(h * W2r_ref[...], axis=-1, keepdims=True) + b2_ref[...]
    gate = (logit > 0.0).astype(jnp.float32)
    ctx_d = jnp.dot(ctx, W_from_ref[...], preferred_element_type=jnp.float32)
    out_ref[...] = x_ref[...] + gate * (ctx_d + b_from_ref[...])


def kernel(x, cache, W_to, b_to, ln_g, ln_b, W1, b1, W2, b2, W_from, b_from):
    B, S, _ = x.shape
    M = cache.shape[1]
    T = B * S
    x2 = x.reshape(T, D_MODEL)
    b_to2 = b_to.reshape(1, D_CACHE)
    ln_g2 = ln_g.reshape(1, 2 * D_CACHE)
    ln_b2 = ln_b.reshape(1, 2 * D_CACHE)
    b12 = b1.reshape(1, D_MODEL // 2)
    W2r = W2.reshape(1, D_MODEL // 2)
    b22 = b2.reshape(1, 1)
    b_from2 = b_from.reshape(1, D_MODEL)

    full = lambda shape: pl.BlockSpec(shape, lambda *a: (0,) * len(shape))

    xc = pl.pallas_call(
        _proj_kernel,
        in_specs=[full((T, D_MODEL)), full((D_MODEL, D_CACHE)),
                  full((1, D_CACHE))],
        out_specs=full((T, D_CACHE)),
        out_shape=jax.ShapeDtypeStruct((T, D_CACHE), jnp.float32),
    )(x2, W_to, b_to2)

    ctx = pl.pallas_call(
        _attend_kernel,
        grid=(B // ATTEND_BATCHES,),
        in_specs=[
            pl.BlockSpec((ATTEND_BATCHES, S, D_CACHE), lambda b: (b, 0, 0)),
            pl.BlockSpec((ATTEND_BATCHES, M, D_CACHE), lambda b: (b, 0, 0)),
        ],
        out_specs=pl.BlockSpec((ATTEND_BATCHES, S, D_CACHE), lambda b: (b, 0, 0)),
        out_shape=jax.ShapeDtypeStruct((B, S, D_CACHE), jnp.float32),
        compiler_params=pltpu.CompilerParams(
            dimension_semantics=("parallel",)),
    )(xc.reshape(B, S, D_CACHE), cache)

    out = pl.pallas_call(
        _gate_kernel,
        in_specs=[full((T, D_MODEL)), full((T, D_CACHE)), full((T, D_CACHE)),
                  full((1, 2 * D_CACHE)), full((1, 2 * D_CACHE)),
                  full((2 * D_CACHE, D_MODEL // 2)), full((1, D_MODEL // 2)),
                  full((1, D_MODEL // 2)), full((1, 1)),
                  full((D_CACHE, D_MODEL)), full((1, D_MODEL))],
        out_specs=full((T, D_MODEL)),
        out_shape=jax.ShapeDtypeStruct((T, D_MODEL), jnp.float32),
    )(x2, xc, ctx.reshape(T, D_CACHE), ln_g2, ln_b2,
      W1, b12, W2r, b22, W_from, b_from2)
    return out.reshape(B, S, D_MODEL)


# speedup vs baseline: 1.3434x; 1.0398x over previous
import math

import jax
import jax.numpy as jnp
from jax.experimental import pallas as pl
from jax.experimental.pallas import tpu as pltpu

D_MODEL = 1024
D_CACHE = 256
INV_SQRT_DC = 1.0 / math.sqrt(D_CACHE)
STEP_BATCHES = 4


def _router_kernel(x_ref, cache_ref, W_to_ref, b_to_ref, ln_g_ref, ln_b_ref,
                   W1_ref, b1_ref, W2r_ref, b2_ref, W_from_ref, b_from_ref,
                   out_ref):
    S = x_ref.shape[1]
    xg = x_ref[...].reshape(STEP_BATCHES * S, D_MODEL)
    xcg = (jnp.dot(xg, W_to_ref[...], preferred_element_type=jnp.float32)
           + b_to_ref[...])

    ctxs = []
    for i in range(STEP_BATCHES):
        xcb = xcg[i * S:(i + 1) * S]
        cb = cache_ref[i]
        scores = jax.lax.dot_general(
            xcb, cb, (((1,), (1,)), ((), ())),
            preferred_element_type=jnp.float32)
        u = jnp.exp(jnp.clip(scores * INV_SQRT_DC, -20.0, 20.0))
        ctx_u = jnp.dot(u, cb, preferred_element_type=jnp.float32)
        denom = jnp.sum(u, axis=-1, keepdims=True)
        ctxs.append(ctx_u / denom)
    ctx = jnp.concatenate(ctxs, axis=0)

    comb = jnp.concatenate([xcg, ctx], axis=-1)
    mean = jnp.mean(comb, axis=-1, keepdims=True)
    var = jnp.mean((comb - mean) ** 2, axis=-1, keepdims=True)
    h = ln_g_ref[...] * (comb - mean) / jnp.sqrt(var + 1e-5) + ln_b_ref[...]
    h = jnp.dot(h, W1_ref[...], preferred_element_type=jnp.float32) + b1_ref[...]
    h = h * jax.nn.sigmoid(h)
    logit = jnp.sum(h * W2r_ref[...], axis=-1, keepdims=True) + b2_ref[...]
    gate = (logit > 0.0).astype(jnp.float32)
    ctx_d = jnp.dot(ctx, W_from_ref[...], preferred_element_type=jnp.float32)
    out = xg + gate * (ctx_d + b_from_ref[...])
    out_ref[...] = out.reshape(STEP_BATCHES, S, D_MODEL)


def kernel(x, cache, W_to, b_to, ln_g, ln_b, W1, b1, W2, b2, W_from, b_from):
    B, S, _ = x.shape
    M = cache.shape[1]
    b_to2 = b_to.reshape(1, D_CACHE)
    ln_g2 = ln_g.reshape(1, 2 * D_CACHE)
    ln_b2 = ln_b.reshape(1, 2 * D_CACHE)
    b12 = b1.reshape(1, D_MODEL // 2)
    W2r = W2.reshape(1, D_MODEL // 2)
    b22 = b2.reshape(1, 1)
    b_from2 = b_from.reshape(1, D_MODEL)

    rep = lambda shape: pl.BlockSpec(shape, lambda b: (0,) * len(shape))
    out = pl.pallas_call(
        _router_kernel,
        grid=(B // STEP_BATCHES,),
        in_specs=[
            pl.BlockSpec((STEP_BATCHES, S, D_MODEL), lambda b: (b, 0, 0)),
            pl.BlockSpec((STEP_BATCHES, M, D_CACHE), lambda b: (b, 0, 0)),
            rep((D_MODEL, D_CACHE)),
            rep((1, D_CACHE)),
            rep((1, 2 * D_CACHE)),
            rep((1, 2 * D_CACHE)),
            rep((2 * D_CACHE, D_MODEL // 2)),
            rep((1, D_MODEL // 2)),
            rep((1, D_MODEL // 2)),
            rep((1, 1)),
            rep((D_CACHE, D_MODEL)),
            rep((1, D_MODEL)),
        ],
        out_specs=pl.BlockSpec((STEP_BATCHES, S, D_MODEL), lambda b: (b, 0, 0)),
        out_shape=jax.ShapeDtypeStruct((B, S, D_MODEL), jnp.float32),
        compiler_params=pltpu.CompilerParams(
            dimension_semantics=("arbitrary",)),
    )(x, cache, W_to, b_to2, ln_g2, ln_b2, W1, b12, W2r, b22, W_from, b_from2)
    return out
